# Initial kernel scaffold; baseline (speedup 1.0000x reference)
import jax
import jax.numpy as jnp
from jax.experimental import pallas as pl


def kernel(h, h_cache, key_pe, Wq, Wk, Wv, Wo, ln1_g, ln1_b, lnm_g, lnm_b, ln2_g, ln2_b, Wg, bg, W1, b1, W2, b2):
    raise NotImplementedError("write your pallas kernel here")


# speedup vs baseline: 27.5041x; 27.5041x over previous
import functools
import math

import jax
import jax.numpy as jnp
from jax.experimental import pallas as pl


_EPS = 1e-5


def _ln(y, g, b):
    mu = jnp.mean(y, axis=-1, keepdims=True)
    d = y - mu
    var = jnp.mean(d * d, axis=-1, keepdims=True)
    return d * jax.lax.rsqrt(var + _EPS) * g + b


def _dot_t(x, w):
    return jax.lax.dot_general(
        x, w, (((1,), (1,)), ((), ())), preferred_element_type=jnp.float32)


def _dot(x, w):
    return jax.lax.dot_general(
        x, w, (((1,), (0,)), ((), ())), preferred_element_type=jnp.float32)


def _proj_kernel(x_ref, w_ref, o_ref):
    o_ref[...] = _dot_t(x_ref[...], w_ref[...])


def _proj(x, w, row_blk):
    T, H = x.shape
    N = w.shape[0]
    return pl.pallas_call(
        _proj_kernel,
        grid=(T // row_blk,),
        in_specs=[
            pl.BlockSpec((row_blk, H), lambda i: (i, 0)),
            pl.BlockSpec((N, H), lambda i: (0, 0)),
        ],
        out_specs=pl.BlockSpec((row_blk, N), lambda i: (i, 0)),
        out_shape=jax.ShapeDtypeStruct((T, N), jnp.float32),
    )(x, w)


def _attn_kernel(q_ref, k_ref, v_ref, pe_ref, o_ref, *, bm, L, win):
    m0 = pl.program_id(1) * bm
    q = q_ref[0]
    kwin = k_ref[0, pl.ds(m0, win), :]
    vwin = v_ref[0, pl.ds(m0, win), :]
    scale = 1.0 / math.sqrt(q.shape[-1])

    t = _dot_t(q, kwin)
    r = _dot(q, pe_ref[...])
    r = jnp.concatenate([r, jnp.zeros((bm, win - L), jnp.float32)], axis=1)
    row = jax.lax.broadcasted_iota(jnp.int32, (bm, win), 0)
    shift = 1
    while shift < bm:
        r = jnp.where((row & shift) != 0, jnp.roll(r, shift, axis=1), r)
        shift *= 2
    col = jax.lax.broadcasted_iota(jnp.int32, (bm, win), 1)
    valid = (col >= row) & (col < row + L)
    s = jnp.where(valid, (t + r) * scale, -1e30)
    m = jnp.max(s, axis=1, keepdims=True)
    p = jnp.exp(s - m)
    p = jnp.where(valid, p, 0.0)
    p = p / jnp.sum(p, axis=1, keepdims=True)
    o_ref[0] = _dot(p, vwin)


def _attention(q, k, v, pe, bm):
    BK, M, D = q.shape
    S = k.shape[1]
    L = pe.shape[1]
    win = L + bm
    kern = functools.partial(_attn_kernel, bm=bm, L=L, win=win)
    return pl.pallas_call(
        kern,
        grid=(BK, M // bm),
        in_specs=[
            pl.BlockSpec((1, bm, D), lambda h, i: (h, i, 0)),
            pl.BlockSpec((1, S, D), lambda h, i: (h, 0, 0)),
            pl.BlockSpec((1, S, D), lambda h, i: (h, 0, 0)),
            pl.BlockSpec((D, L), lambda h, i: (0, 0)),
        ],
        out_specs=pl.BlockSpec((1, bm, D), lambda h, i: (h, i, 0)),
        out_shape=jax.ShapeDtypeStruct((BK, M, D), jnp.float32),
    )(q, k, v, pe)


def _wo_ln_kernel(x_ref, h_ref, w_ref, g_ref, b_ref, o_ref):
    y = h_ref[...] + _dot_t(x_ref[...], w_ref[...])
    o_ref[...] = _ln(y, g_ref[...], b_ref[...])


def _wo_ln(x, h, w, g, b, row_blk):
    T, H = x.shape
    return pl.pallas_call(
        _wo_ln_kernel,
        grid=(T // row_blk,),
        in_specs=[
            pl.BlockSpec((row_blk, H), lambda i: (i, 0)),
            pl.BlockSpec((row_blk, H), lambda i: (i, 0)),
            pl.BlockSpec((H, H), lambda i: (0, 0)),
            pl.BlockSpec((1, H), lambda i: (0, 0)),
            pl.BlockSpec((1, H), lambda i: (0, 0)),
        ],
        out_specs=pl.BlockSpec((row_blk, H), lambda i: (i, 0)),
        out_shape=jax.ShapeDtypeStruct((T, H), jnp.float32),
    )(x, h, w, g, b)


def _router_kernel(x_ref, wg_ref, bg_ref, o_ref):
    logits = _dot_t(x_ref[...], wg_ref[...]) + bg_ref[...]
    R, E = logits.shape
    e_iota = jax.lax.broadcasted_iota(jnp.int32, (R, E), 1)
    m1 = jnp.max(logits, axis=1, keepdims=True)
    i1 = jnp.min(jnp.where(logits == m1, e_iota, E), axis=1, keepdims=True)
    masked = jnp.where(e_iota == i1, -jnp.inf, logits)
    m2 = jnp.max(masked, axis=1, keepdims=True)
    i2 = jnp.min(jnp.where(masked == m2, e_iota, E), axis=1, keepdims=True)
    p1 = 1.0 / (1.0 + jnp.exp(m2 - m1))
    p2 = 1.0 - p1
    o_ref[...] = (jnp.where(e_iota == i1, p1, 0.0)
                  + jnp.where(e_iota == i2, p2, 0.0))


def _router(x, wg, bg, row_blk):
    T, H = x.shape
    E = wg.shape[0]
    return pl.pallas_call(
        _router_kernel,
        grid=(T // row_blk,),
        in_specs=[
            pl.BlockSpec((row_blk, H), lambda i: (i, 0)),
            pl.BlockSpec((E, H), lambda i: (0, 0)),
            pl.BlockSpec((1, E), lambda i: (0, 0)),
        ],
        out_specs=pl.BlockSpec((row_blk, E), lambda i: (i, 0)),
        out_shape=jax.ShapeDtypeStruct((T, E), jnp.float32),
    )(x, wg, bg)


def _moe_kernel(x_ref, w1_ref, b1_ref, w2_ref, b2_ref, wt_ref,
                lnm_g_ref, lnm_b_ref, ln2_g_ref, ln2_b_ref, o_ref,
                *, n_e, n_f):
    e = pl.program_id(1)
    f = pl.program_id(2)
    x = x_ref[...]
    he = jnp.maximum(_dot_t(x, w1_ref[0]) + b1_ref[0], 0.0)
    contrib = _dot_t(he, w2_ref[0]) * wt_ref[0]
    contrib = jnp.where(f == 0, contrib + wt_ref[0] * b2_ref[0], contrib)

    @pl.when((e == 0) & (f == 0))
    def _():
        o_ref[...] = contrib

    @pl.when((e > 0) | (f > 0))
    def _():
        o_ref[...] += contrib

    @pl.when((e == n_e - 1) & (f == n_f - 1))
    def _():
        h1 = x
        core = o_ref[...]
        smoe = _ln(h1 + core, lnm_g_ref[...], lnm_b_ref[...])
        o_ref[...] = _ln(h1 + smoe, ln2_g_ref[...], ln2_b_ref[...])


def _moe(x, w1, b1, w2, b2, wt, lnm_g, lnm_b, ln2_g, ln2_b, row_blk, ff_blk):
    T, H = x.shape
    E, FF, _ = w1.shape
    n_t, n_f = T // row_blk, FF // ff_blk
    kern = functools.partial(_moe_kernel, n_e=E, n_f=n_f)
    return pl.pallas_call(
        kern,
        grid=(n_t, E, n_f),
        in_specs=[
            pl.BlockSpec((row_blk, H), lambda t, e, f: (t, 0)),
            pl.BlockSpec((1, ff_blk, H), lambda t, e, f: (e, f, 0)),
            pl.BlockSpec((1, 1, ff_blk), lambda t, e, f: (e, 0, f)),
            pl.BlockSpec((1, H, ff_blk), lambda t, e, f: (e, 0, f)),
            pl.BlockSpec((1, 1, H), lambda t, e, f: (e, 0, 0)),
            pl.BlockSpec((1, row_blk, 1), lambda t, e, f: (e, t, 0)),
            pl.BlockSpec((1, H), lambda t, e, f: (0, 0)),
            pl.BlockSpec((1, H), lambda t, e, f: (0, 0)),
            pl.BlockSpec((1, H), lambda t, e, f: (0, 0)),
            pl.BlockSpec((1, H), lambda t, e, f: (0, 0)),
        ],
        out_specs=pl.BlockSpec((row_blk, H), lambda t, e, f: (t, 0)),
        out_shape=jax.ShapeDtypeStruct((T, H), jnp.float32),
    )(x, w1, b1, w2, b2, wt, lnm_g, lnm_b, ln2_g, ln2_b)


def kernel(h, h_cache, key_pe, Wq, Wk, Wv, Wo, ln1_g, ln1_b, lnm_g, lnm_b,
           ln2_g, ln2_b, Wg, bg, W1, b1, W2, b2):
    B, M, H = h.shape
    L = h_cache.shape[1]
    D = key_pe.shape[1]
    K = H // D
    S = L + M
    E, FF, _ = W1.shape
    T = B * M

    h_all = jnp.concatenate([h_cache, h], axis=1)
    pe = key_pe.reshape(D, L)

    q_flat = _proj(h.reshape(T, H), Wq, min(1024, T))
    wkv = jnp.concatenate([Wk, Wv], axis=0)
    kv_flat = _proj(h_all.reshape(B * S, H), wkv, min(1024, B * S))

    def heads(x, Bx, Tx):
        return (x.reshape(Bx, Tx, K, D).transpose(0, 2, 1, 3)
                .reshape(Bx * K, Tx, D))

    q = heads(q_flat, B, M)
    k = heads(kv_flat[:, :H], B, S)
    v = heads(kv_flat[:, H:], B, S)

    attn = _attention(q, k, v, pe, bm=128)
    attn = (attn.reshape(B, K, M, D).transpose(0, 2, 1, 3).reshape(T, H))

    h1 = _wo_ln(attn, h.reshape(T, H), Wo,
                ln1_g.reshape(1, H), ln1_b.reshape(1, H), min(1024, T))

    wt = _router(h1, Wg, bg.reshape(1, E), min(1024, T))
    wt = wt.T.reshape(E, T, 1)

    h2 = _moe(h1, W1, b1.reshape(E, 1, FF), W2, b2.reshape(E, 1, H), wt,
              lnm_g.reshape(1, H), lnm_b.reshape(1, H),
              ln2_g.reshape(1, H), ln2_b.reshape(1, H),
              row_blk=min(1024, T), ff_blk=min(768, FF))
    return h2.reshape(B, M, H)


# speedup vs baseline: 27.8794x; 1.0136x over previous
import functools
import math

import jax
import jax.numpy as jnp
from jax.experimental import pallas as pl


_EPS = 1e-5


def _ln(y, g, b):
    mu = jnp.mean(y, axis=-1, keepdims=True)
    d = y - mu
    var = jnp.mean(d * d, axis=-1, keepdims=True)
    return d * jax.lax.rsqrt(var + _EPS) * g + b


def _dot_t(x, w):
    return jax.lax.dot_general(
        x, w, (((1,), (1,)), ((), ())), preferred_element_type=jnp.float32)


def _dot(x, w):
    return jax.lax.dot_general(
        x, w, (((1,), (0,)), ((), ())), preferred_element_type=jnp.float32)


def _bf(x):
    return x.astype(jnp.bfloat16)


def _dot_t16(x, w):
    return jax.lax.dot_general(
        _bf(x), _bf(w), (((1,), (1,)), ((), ())),
        preferred_element_type=jnp.float32)


def _dot16(x, w):
    return jax.lax.dot_general(
        _bf(x), _bf(w), (((1,), (0,)), ((), ())),
        preferred_element_type=jnp.float32)


def _proj_kernel(x_ref, w_ref, o_ref):
    o_ref[...] = _bf(_dot_t16(x_ref[...], w_ref[...]))


def _proj(x, w, row_blk):
    T, H = x.shape
    N = w.shape[0]
    return pl.pallas_call(
        _proj_kernel,
        grid=(T // row_blk,),
        in_specs=[
            pl.BlockSpec((row_blk, H), lambda i: (i, 0)),
            pl.BlockSpec((N, H), lambda i: (0, 0)),
        ],
        out_specs=pl.BlockSpec((row_blk, N), lambda i: (i, 0)),
        out_shape=jax.ShapeDtypeStruct((T, N), jnp.bfloat16),
    )(x, w)


def _attn_kernel(q_ref, k_ref, v_ref, pe_ref, o_ref, *, bm, L, win):
    m0 = pl.program_id(1) * bm
    q = q_ref[0]
    kwin = k_ref[0, pl.ds(m0, win), :]
    vwin = v_ref[0, pl.ds(m0, win), :]
    scale = 1.0 / math.sqrt(q.shape[-1])

    t = _dot_t16(q, kwin)
    r = _dot16(q, pe_ref[...])
    r = jnp.concatenate([r, jnp.zeros((bm, win - L), jnp.float32)], axis=1)
    row = jax.lax.broadcasted_iota(jnp.int32, (bm, win), 0)
    shift = 1
    while shift < bm:
        r = jnp.where((row & shift) != 0, jnp.roll(r, shift, axis=1), r)
        shift *= 2
    col = jax.lax.broadcasted_iota(jnp.int32, (bm, win), 1)
    valid = (col >= row) & (col < row + L)
    s = jnp.where(valid, (t + r) * scale, -1e30)
    m = jnp.max(s, axis=1, keepdims=True)
    p = jnp.exp(s - m)
    p = jnp.where(valid, p, 0.0)
    p = p / jnp.sum(p, axis=1, keepdims=True)
    o_ref[0] = _bf(_dot16(p, vwin))


def _attention(q, k, v, pe, bm):
    BK, M, D = q.shape
    S = k.shape[1]
    L = pe.shape[1]
    win = L + bm
    kern = functools.partial(_attn_kernel, bm=bm, L=L, win=win)
    return pl.pallas_call(
        kern,
        grid=(BK, M // bm),
        in_specs=[
            pl.BlockSpec((1, bm, D), lambda h, i: (h, i, 0)),
            pl.BlockSpec((1, S, D), lambda h, i: (h, 0, 0)),
            pl.BlockSpec((1, S, D), lambda h, i: (h, 0, 0)),
            pl.BlockSpec((D, L), lambda h, i: (0, 0)),
        ],
        out_specs=pl.BlockSpec((1, bm, D), lambda h, i: (h, i, 0)),
        out_shape=jax.ShapeDtypeStruct((BK, M, D), jnp.bfloat16),
    )(q, k, v, pe)


def _wo_ln_kernel(x_ref, h_ref, w_ref, g_ref, b_ref, o_ref):
    y = h_ref[...] + _dot_t16(x_ref[...], w_ref[...])
    o_ref[...] = _ln(y, g_ref[...], b_ref[...])


def _wo_ln(x, h, w, g, b, row_blk):
    T, H = x.shape
    return pl.pallas_call(
        _wo_ln_kernel,
        grid=(T // row_blk,),
        in_specs=[
            pl.BlockSpec((row_blk, H), lambda i: (i, 0)),
            pl.BlockSpec((row_blk, H), lambda i: (i, 0)),
            pl.BlockSpec((H, H), lambda i: (0, 0)),
            pl.BlockSpec((1, H), lambda i: (0, 0)),
            pl.BlockSpec((1, H), lambda i: (0, 0)),
        ],
        out_specs=pl.BlockSpec((row_blk, H), lambda i: (i, 0)),
        out_shape=jax.ShapeDtypeStruct((T, H), jnp.float32),
    )(x, h, w, g, b)


def _router_kernel(x_ref, wg_ref, bg_ref, o_ref):
    logits = _dot_t(x_ref[...], wg_ref[...]) + bg_ref[...]
    R, E = logits.shape
    e_iota = jax.lax.broadcasted_iota(jnp.int32, (R, E), 1)
    m1 = jnp.max(logits, axis=1, keepdims=True)
    i1 = jnp.min(jnp.where(logits == m1, e_iota, E), axis=1, keepdims=True)
    masked = jnp.where(e_iota == i1, -jnp.inf, logits)
    m2 = jnp.max(masked, axis=1, keepdims=True)
    i2 = jnp.min(jnp.where(masked == m2, e_iota, E), axis=1, keepdims=True)
    p1 = 1.0 / (1.0 + jnp.exp(m2 - m1))
    p2 = 1.0 - p1
    o_ref[...] = (jnp.where(e_iota == i1, p1, 0.0)
                  + jnp.where(e_iota == i2, p2, 0.0))


def _router(x, wg, bg, row_blk):
    T, H = x.shape
    E = wg.shape[0]
    return pl.pallas_call(
        _router_kernel,
        grid=(T // row_blk,),
        in_specs=[
            pl.BlockSpec((row_blk, H), lambda i: (i, 0)),
            pl.BlockSpec((E, H), lambda i: (0, 0)),
            pl.BlockSpec((1, E), lambda i: (0, 0)),
        ],
        out_specs=pl.BlockSpec((row_blk, E), lambda i: (i, 0)),
        out_shape=jax.ShapeDtypeStruct((T, E), jnp.float32),
    )(x, wg, bg)


def _moe_kernel(x_ref, w1_ref, b1_ref, w2_ref, b2_ref, wt_ref,
                lnm_g_ref, lnm_b_ref, ln2_g_ref, ln2_b_ref, o_ref,
                *, n_e, n_f):
    e = pl.program_id(1)
    f = pl.program_id(2)
    x = x_ref[...]
    he = jnp.maximum(_dot_t16(x, w1_ref[0]) + b1_ref[0], 0.0)
    contrib = _dot_t16(he, w2_ref[0]) * wt_ref[0]
    contrib = jnp.where(f == 0, contrib + wt_ref[0] * b2_ref[0], contrib)

    @pl.when((e == 0) & (f == 0))
    def _():
        o_ref[...] = contrib

    @pl.when((e > 0) | (f > 0))
    def _():
        o_ref[...] += contrib

    @pl.when((e == n_e - 1) & (f == n_f - 1))
    def _():
        h1 = x
        core = o_ref[...]
        smoe = _ln(h1 + core, lnm_g_ref[...], lnm_b_ref[...])
        o_ref[...] = _ln(h1 + smoe, ln2_g_ref[...], ln2_b_ref[...])


def _moe(x, w1, b1, w2, b2, wt, lnm_g, lnm_b, ln2_g, ln2_b, row_blk, ff_blk):
    T, H = x.shape
    E, FF, _ = w1.shape
    n_t, n_f = T // row_blk, FF // ff_blk
    kern = functools.partial(_moe_kernel, n_e=E, n_f=n_f)
    return pl.pallas_call(
        kern,
        grid=(n_t, E, n_f),
        in_specs=[
            pl.BlockSpec((row_blk, H), lambda t, e, f: (t, 0)),
            pl.BlockSpec((1, ff_blk, H), lambda t, e, f: (e, f, 0)),
            pl.BlockSpec((1, 1, ff_blk), lambda t, e, f: (e, 0, f)),
            pl.BlockSpec((1, H, ff_blk), lambda t, e, f: (e, 0, f)),
            pl.BlockSpec((1, 1, H), lambda t, e, f: (e, 0, 0)),
            pl.BlockSpec((1, row_blk, 1), lambda t, e, f: (e, t, 0)),
            pl.BlockSpec((1, H), lambda t, e, f: (0, 0)),
            pl.BlockSpec((1, H), lambda t, e, f: (0, 0)),
            pl.BlockSpec((1, H), lambda t, e, f: (0, 0)),
            pl.BlockSpec((1, H), lambda t, e, f: (0, 0)),
        ],
        out_specs=pl.BlockSpec((row_blk, H), lambda t, e, f: (t, 0)),
        out_shape=jax.ShapeDtypeStruct((T, H), jnp.float32),
    )(x, w1, b1, w2, b2, wt, lnm_g, lnm_b, ln2_g, ln2_b)


def kernel(h, h_cache, key_pe, Wq, Wk, Wv, Wo, ln1_g, ln1_b, lnm_g, lnm_b,
           ln2_g, ln2_b, Wg, bg, W1, b1, W2, b2):
    B, M, H = h.shape
    L = h_cache.shape[1]
    D = key_pe.shape[1]
    K = H // D
    S = L + M
    E, FF, _ = W1.shape
    T = B * M

    h_all = jnp.concatenate([h_cache, h], axis=1)
    pe = _bf(key_pe.reshape(D, L))

    q_flat = _proj(h.reshape(T, H), _bf(Wq), min(1024, T))
    wkv = _bf(jnp.concatenate([Wk, Wv], axis=0))
    kv_flat = _proj(h_all.reshape(B * S, H), wkv, min(1024, B * S))

    def heads(x, Bx, Tx):
        return (x.reshape(Bx, Tx, K, D).transpose(0, 2, 1, 3)
                .reshape(Bx * K, Tx, D))

    q = heads(q_flat, B, M)
    k = heads(kv_flat[:, :H], B, S)
    v = heads(kv_flat[:, H:], B, S)

    attn = _attention(q, k, v, pe, bm=128)
    attn = (attn.reshape(B, K, M, D).transpose(0, 2, 1, 3).reshape(T, H))

    h1 = _wo_ln(attn, h.reshape(T, H), _bf(Wo),
                ln1_g.reshape(1, H), ln1_b.reshape(1, H), min(1024, T))

    wt = _router(h1, Wg, bg.reshape(1, E), min(1024, T))
    wt = wt.T.reshape(E, T, 1)

    h2 = _moe(h1, _bf(W1), b1.reshape(E, 1, FF), _bf(W2), b2.reshape(E, 1, H),
              wt, lnm_g.reshape(1, H), lnm_b.reshape(1, H),
              ln2_g.reshape(1, H), ln2_b.reshape(1, H),
              row_blk=min(2048, T), ff_blk=min(768, FF))
    return h2.reshape(B, M, H)


# speedup vs baseline: 35.2575x; 1.2646x over previous
import functools
import math

import jax
import jax.numpy as jnp
from jax import lax
from jax.experimental import pallas as pl
from jax.experimental.pallas import tpu as pltpu
from jax.experimental.pallas import tpu_sc as plsc

_SC_CORES = 2
_SC_SUBCORES = 16


_EPS = 1e-5


def _ln(y, g, b):
    mu = jnp.mean(y, axis=-1, keepdims=True)
    d = y - mu
    var = jnp.mean(d * d, axis=-1, keepdims=True)
    return d * jax.lax.rsqrt(var + _EPS) * g + b


def _dot_t(x, w):
    return jax.lax.dot_general(
        x, w, (((1,), (1,)), ((), ())), preferred_element_type=jnp.float32)


def _dot(x, w):
    return jax.lax.dot_general(
        x, w, (((1,), (0,)), ((), ())), preferred_element_type=jnp.float32)


def _bf(x):
    return x.astype(jnp.bfloat16)


def _dot_t16(x, w):
    return jax.lax.dot_general(
        _bf(x), _bf(w), (((1,), (1,)), ((), ())),
        preferred_element_type=jnp.float32)


def _dot16(x, w):
    return jax.lax.dot_general(
        _bf(x), _bf(w), (((1,), (0,)), ((), ())),
        preferred_element_type=jnp.float32)


def _proj_kernel(x_ref, w_ref, o_ref):
    o_ref[...] = _bf(_dot_t16(x_ref[...], w_ref[...]))


def _proj(x, w, row_blk):
    T, H = x.shape
    N = w.shape[0]
    return pl.pallas_call(
        _proj_kernel,
        grid=(T // row_blk,),
        in_specs=[
            pl.BlockSpec((row_blk, H), lambda i: (i, 0)),
            pl.BlockSpec((N, H), lambda i: (0, 0)),
        ],
        out_specs=pl.BlockSpec((row_blk, N), lambda i: (i, 0)),
        out_shape=jax.ShapeDtypeStruct((T, N), jnp.bfloat16),
    )(x, w)


def _attn_kernel(q_ref, k_ref, v_ref, pe_ref, msk_ref, o_ref, *, bm, win):
    m0 = pl.program_id(1) * bm
    q = (q_ref[0].astype(jnp.float32) * (1.0 / math.sqrt(q_ref.shape[-1])))
    kwin = k_ref[0, pl.ds(m0, win), :]
    vwin = v_ref[0, pl.ds(m0, win), :]

    t = _bf(_dot_t16(q, kwin))
    r = _bf(_dot16(q, pe_ref[...]))
    row = jax.lax.broadcasted_iota(jnp.int16, (bm, win), 0)
    shift = 1
    while shift < bm:
        r = jnp.where((row & jnp.int16(shift)) != 0,
                      jnp.roll(r, shift, axis=1), r)
        shift *= 2
    s = t + r + msk_ref[...]
    m = jnp.max(s, axis=1, keepdims=True)
    p = jnp.exp(s - m)
    denom = jnp.sum(p.astype(jnp.float32), axis=1, keepdims=True)
    o_ref[0] = _bf(_dot16(p, vwin) / denom)


def _attention(q, k, v, pe, bm):
    BK, M, D = q.shape
    S = k.shape[1]
    L = pe.shape[1]
    win = L + bm
    pe_pad = jnp.concatenate(
        [pe, jnp.zeros((D, win - L), pe.dtype)], axis=1)
    ii = jnp.arange(bm)[:, None]
    jj = jnp.arange(win)[None, :]
    msk = jnp.where((jj >= ii) & (jj < ii + L),
                    0.0, -jnp.inf).astype(jnp.bfloat16)
    kern = functools.partial(_attn_kernel, bm=bm, win=win)
    return pl.pallas_call(
        kern,
        grid=(BK, M // bm),
        in_specs=[
            pl.BlockSpec((1, bm, D), lambda h, i: (h, i, 0)),
            pl.BlockSpec((1, S, D), lambda h, i: (h, 0, 0)),
            pl.BlockSpec((1, S, D), lambda h, i: (h, 0, 0)),
            pl.BlockSpec((D, win), lambda h, i: (0, 0)),
            pl.BlockSpec((bm, win), lambda h, i: (0, 0)),
        ],
        out_specs=pl.BlockSpec((1, bm, D), lambda h, i: (h, i, 0)),
        out_shape=jax.ShapeDtypeStruct((BK, M, D), jnp.bfloat16),
    )(q, k, v, pe_pad, msk)


def _wo_ln_kernel(x_ref, h_ref, w_ref, g_ref, b_ref, o_ref):
    y = h_ref[...] + _dot_t16(x_ref[...], w_ref[...])
    o_ref[...] = _ln(y, g_ref[...], b_ref[...])


def _wo_ln(x, h, w, g, b, row_blk):
    T, H = x.shape
    return pl.pallas_call(
        _wo_ln_kernel,
        grid=(T // row_blk,),
        in_specs=[
            pl.BlockSpec((row_blk, H), lambda i: (i, 0)),
            pl.BlockSpec((row_blk, H), lambda i: (i, 0)),
            pl.BlockSpec((H, H), lambda i: (0, 0)),
            pl.BlockSpec((1, H), lambda i: (0, 0)),
            pl.BlockSpec((1, H), lambda i: (0, 0)),
        ],
        out_specs=pl.BlockSpec((row_blk, H), lambda i: (i, 0)),
        out_shape=jax.ShapeDtypeStruct((T, H), jnp.float32),
    )(x, h, w, g, b)


def _router_kernel(x_ref, wg_ref, bg_ref, idx_ref, prb_ref):
    logits = _dot_t(x_ref[...], wg_ref[...]) + bg_ref[...]
    R, E = logits.shape
    e_iota = jax.lax.broadcasted_iota(jnp.int32, (R, E), 1)
    m1 = jnp.max(logits, axis=1, keepdims=True)
    i1 = jnp.min(jnp.where(logits == m1, e_iota, E), axis=1, keepdims=True)
    masked = jnp.where(e_iota == i1, -jnp.inf, logits)
    m2 = jnp.max(masked, axis=1, keepdims=True)
    i2 = jnp.min(jnp.where(masked == m2, e_iota, E), axis=1, keepdims=True)
    p1 = 1.0 / (1.0 + jnp.exp(m2 - m1))
    p2 = 1.0 - p1
    idx_ref[...] = jnp.concatenate([i1, i2], axis=1)
    prb_ref[...] = jnp.concatenate([p1, p2], axis=1)


def _router(x, wg, bg, row_blk):
    T, H = x.shape
    E = wg.shape[0]
    return pl.pallas_call(
        _router_kernel,
        grid=(T // row_blk,),
        in_specs=[
            pl.BlockSpec((row_blk, H), lambda i: (i, 0)),
            pl.BlockSpec((E, H), lambda i: (0, 0)),
            pl.BlockSpec((1, E), lambda i: (0, 0)),
        ],
        out_specs=[pl.BlockSpec((row_blk, 2), lambda i: (i, 0)),
                   pl.BlockSpec((row_blk, 2), lambda i: (i, 0))],
        out_shape=[jax.ShapeDtypeStruct((T, 2), jnp.int32),
                   jax.ShapeDtypeStruct((T, 2), jnp.float32)],
    )(x, wg, bg)


def _rank_kernel(ids_ref, rank_ref, cnt_ref, carry, *, n_e, blk):
    @pl.when(pl.program_id(0) == 0)
    def _():
        carry[...] = jnp.zeros_like(carry)

    ids = ids_ref[...]
    e_iota = jax.lax.broadcasted_iota(jnp.int32, (blk, n_e), 1)
    oh = (ids == e_iota).astype(jnp.int32)
    inc = oh
    sh = 1
    while sh < blk:
        inc = inc + jnp.concatenate(
            [jnp.zeros((sh, n_e), jnp.int32), inc[:-sh]], axis=0)
        sh *= 2
    rank_e = (inc - oh) + carry[...]
    rank_ref[...] = jnp.sum(oh * rank_e, axis=1, keepdims=True)
    carry[...] += jnp.sum(oh, axis=0, keepdims=True)
    cnt_ref[...] = carry[...]


def _rank(ids, n_e, blk):
    T2 = ids.shape[0]
    kern = functools.partial(_rank_kernel, n_e=n_e, blk=blk)
    return pl.pallas_call(
        kern,
        grid=(T2 // blk,),
        in_specs=[pl.BlockSpec((blk, 1), lambda i: (i, 0))],
        out_specs=[pl.BlockSpec((blk, 1), lambda i: (i, 0)),
                   pl.BlockSpec((1, n_e), lambda i: (0, 0))],
        out_shape=[jax.ShapeDtypeStruct((T2, 1), jnp.int32),
                   jax.ShapeDtypeStruct((1, n_e), jnp.int32)],
        scratch_shapes=[pltpu.VMEM((1, n_e), jnp.int32)],
    )(ids)


def _pos_kernel(ids_ref, rank_ref, off_ref, pos_ref, *, n_e, blk):
    ids = ids_ref[...]
    e_iota = jax.lax.broadcasted_iota(jnp.int32, (blk, n_e), 1)
    oh = (ids == e_iota).astype(jnp.int32)
    base = jnp.sum(oh * off_ref[...], axis=1, keepdims=True)
    pos_ref[...] = base + rank_ref[...]


def _pos(ids, rank, off, n_e, blk):
    T2 = ids.shape[0]
    kern = functools.partial(_pos_kernel, n_e=n_e, blk=blk)
    return pl.pallas_call(
        kern,
        grid=(T2 // blk,),
        in_specs=[pl.BlockSpec((blk, 1), lambda i: (i, 0)),
                  pl.BlockSpec((blk, 1), lambda i: (i, 0)),
                  pl.BlockSpec((1, n_e), lambda i: (0, 0))],
        out_specs=pl.BlockSpec((blk, 1), lambda i: (i, 0)),
        out_shape=jax.ShapeDtypeStruct((T2, 1), jnp.int32),
    )(ids, rank, off)


def _gmm_meta(cnt, T2, bm, n_e):
    tb = T2 // bm
    ni = tb + n_e - 1
    off = jnp.concatenate([jnp.zeros(1, jnp.int32),
                           jnp.cumsum(cnt.reshape(n_e)).astype(jnp.int32)])
    lo_b = (jnp.arange(tb) * bm)[:, None]
    inter = (off[None, :n_e] < lo_b + bm) & (off[None, 1:] > lo_b)
    keep = inter.reshape(-1)
    dest = jnp.cumsum(keep.astype(jnp.int32)) - 1
    dm = jnp.where(keep, dest, ni)
    r_flat = jnp.repeat(jnp.arange(tb, dtype=jnp.int32), n_e)
    e_flat = jnp.tile(jnp.arange(n_e, dtype=jnp.int32), tb)
    z = jnp.zeros(ni + 1, jnp.int32)
    item_r = z.at[dm].set(r_flat)[:ni]
    item_e = z.at[dm].set(e_flat)[:ni]
    first = (inter & (jnp.cumsum(inter.astype(jnp.int32), axis=1) == 1))
    item_first = z.at[dm].set(first.reshape(-1).astype(jnp.int32))[:ni]
    item_valid = z.at[dm].set(1)[:ni]
    n_items = jnp.sum(keep.astype(jnp.int32))
    sl = jnp.arange(ni)
    item_r = jnp.where(sl >= n_items, item_r[n_items - 1], item_r)
    item_e = jnp.where(sl >= n_items, item_e[n_items - 1], item_e)
    return jnp.concatenate([item_r, item_e, item_first, item_valid, off]), ni


def _gmm_kernel(meta_ref, x_ref, w1_ref, b1_ref, w2_ref, b2_ref, o_ref,
                *, ni, bm):
    i = pl.program_id(0)
    f = pl.program_id(1)
    r = meta_ref[i]
    e = meta_ref[ni + i]
    first = meta_ref[2 * ni + i]
    valid = meta_ref[3 * ni + i]
    lo_g = meta_ref[4 * ni + e]
    hi_g = meta_ref[4 * ni + e + 1]
    he = jnp.maximum(_dot_t16(x_ref[...], w1_ref[0]) + b1_ref[0], 0.0)
    contrib = _dot_t16(he, w2_ref[0])
    contrib = jnp.where(f == 0, contrib + b2_ref[0], contrib)
    ridx = jax.lax.broadcasted_iota(jnp.int32, (bm, 1), 0) + r * bm
    msk = (ridx >= lo_g) & (ridx < hi_g) & (valid > 0)
    contrib = jnp.where(msk, contrib, 0.0)

    @pl.when((first > 0) & (f == 0))
    def _():
        o_ref[...] = contrib

    @pl.when((first == 0) | (f > 0))
    def _():
        o_ref[...] += contrib


def _gmm(meta, ni, xg, w1, b1, w2, b2, bm, ffb):
    T2, H = xg.shape
    E, FF, _ = w1.shape
    n_f = FF // ffb
    kern = functools.partial(_gmm_kernel, ni=ni, bm=bm)
    grid_spec = pltpu.PrefetchScalarGridSpec(
        num_scalar_prefetch=1,
        grid=(ni, n_f),
        in_specs=[
            pl.BlockSpec((bm, H), lambda i, f, m: (m[i], 0)),
            pl.BlockSpec((1, ffb, H), lambda i, f, m: (m[ni + i], f, 0)),
            pl.BlockSpec((1, 1, ffb), lambda i, f, m: (m[ni + i], 0, f)),
            pl.BlockSpec((1, H, ffb), lambda i, f, m: (m[ni + i], 0, f)),
            pl.BlockSpec((1, 1, H), lambda i, f, m: (m[ni + i], 0, 0)),
        ],
        out_specs=pl.BlockSpec((bm, H), lambda i, f, m: (m[i], 0)),
    )
    return pl.pallas_call(
        kern,
        grid_spec=grid_spec,
        out_shape=jax.ShapeDtypeStruct((T2, H), jnp.float32),
    )(meta, xg, w1, b1, w2, b2)


def _sc_scatter(x, pos0, pos1):
    T, H = x.shape
    nw = _SC_CORES * _SC_SUBCORES
    tw = T // nw
    mesh = plsc.VectorSubcoreMesh(core_axis_name="c", subcore_axis_name="s")

    @functools.partial(
        pl.kernel, mesh=mesh,
        out_type=jax.ShapeDtypeStruct((2 * T, H), jnp.float32),
        scratch_types=[
            pltpu.VMEM((tw,), jnp.int32),
            pltpu.VMEM((tw,), jnp.int32),
            pltpu.VMEM((tw, H), jnp.float32),
            pltpu.SemaphoreType.DMA,
        ],
    )
    def k(x_hbm, p0_hbm, p1_hbm, out_hbm, i0_v, i1_v, rows_v, sem):
        wid = lax.axis_index("s") * _SC_CORES + lax.axis_index("c")
        base = wid * tw
        pltpu.sync_copy(p0_hbm.at[pl.ds(base, tw)], i0_v)
        pltpu.sync_copy(p1_hbm.at[pl.ds(base, tw)], i1_v)
        pltpu.sync_copy(x_hbm.at[pl.ds(base, tw)], rows_v)
        pltpu.async_copy(rows_v, out_hbm.at[i0_v], sem).wait()
        pltpu.async_copy(rows_v, out_hbm.at[i1_v], sem).wait()

    return k(x, pos0, pos1)


def _sc_gather(yg, pos0, pos1):
    T2, H = yg.shape
    T = T2 // 2
    nw = _SC_CORES * _SC_SUBCORES
    tw = T // nw
    mesh = plsc.VectorSubcoreMesh(core_axis_name="c", subcore_axis_name="s")

    @functools.partial(
        pl.kernel, mesh=mesh,
        out_type=(jax.ShapeDtypeStruct((T, H), jnp.float32),
                  jax.ShapeDtypeStruct((T, H), jnp.float32)),
        scratch_types=[
            pltpu.VMEM((tw,), jnp.int32),
            pltpu.VMEM((tw, H), jnp.float32),
            pltpu.SemaphoreType.DMA,
        ],
    )
    def k(yg_hbm, p0_hbm, p1_hbm, g0_hbm, g1_hbm, idx_v, rows_v, sem):
        wid = lax.axis_index("s") * _SC_CORES + lax.axis_index("c")
        base = wid * tw
        pltpu.sync_copy(p0_hbm.at[pl.ds(base, tw)], idx_v)
        pltpu.async_copy(yg_hbm.at[idx_v], rows_v, sem).wait()
        pltpu.sync_copy(rows_v, g0_hbm.at[pl.ds(base, tw)])
        pltpu.sync_copy(p1_hbm.at[pl.ds(base, tw)], idx_v)
        pltpu.async_copy(yg_hbm.at[idx_v], rows_v, sem).wait()
        pltpu.sync_copy(rows_v, g1_hbm.at[pl.ds(base, tw)])

    return k(yg, pos0, pos1)


def _comb_kernel(h1_ref, g0_ref, g1_ref, p0_ref, p1_ref,
                 lnm_g_ref, lnm_b_ref, ln2_g_ref, ln2_b_ref, o_ref):
    core = p0_ref[...] * g0_ref[...] + p1_ref[...] * g1_ref[...]
    h1 = h1_ref[...]
    smoe = _ln(h1 + core, lnm_g_ref[...], lnm_b_ref[...])
    o_ref[...] = _ln(h1 + smoe, ln2_g_ref[...], ln2_b_ref[...])


def _comb_ln(h1, g0, g1, p0, p1, lnm_g, lnm_b, ln2_g, ln2_b, row_blk):
    T, H = h1.shape
    rb = pl.BlockSpec((row_blk, H), lambda i: (i, 0))
    cb = pl.BlockSpec((row_blk, 1), lambda i: (i, 0))
    vb = pl.BlockSpec((1, H), lambda i: (0, 0))
    return pl.pallas_call(
        _comb_kernel,
        grid=(T // row_blk,),
        in_specs=[rb, rb, rb, cb, cb, vb, vb, vb, vb],
        out_specs=rb,
        out_shape=jax.ShapeDtypeStruct((T, H), jnp.float32),
    )(h1, g0, g1, p0, p1, lnm_g, lnm_b, ln2_g, ln2_b)


def kernel(h, h_cache, key_pe, Wq, Wk, Wv, Wo, ln1_g, ln1_b, lnm_g, lnm_b,
           ln2_g, ln2_b, Wg, bg, W1, b1, W2, b2):
    B, M, H = h.shape
    L = h_cache.shape[1]
    D = key_pe.shape[1]
    K = H // D
    S = L + M
    E, FF, _ = W1.shape
    T = B * M

    h_all = jnp.concatenate([h_cache, h], axis=1)
    pe = _bf(key_pe.reshape(D, L))

    q_flat = _proj(h.reshape(T, H), _bf(Wq), min(1024, T))
    wkv = _bf(jnp.concatenate([Wk, Wv], axis=0))
    kv_flat = _proj(h_all.reshape(B * S, H), wkv, min(1024, B * S))

    def heads(x, Bx, Tx):
        return (x.reshape(Bx, Tx, K, D).transpose(0, 2, 1, 3)
                .reshape(Bx * K, Tx, D))

    q = heads(q_flat, B, M)
    k = heads(kv_flat[:, :H], B, S)
    v = heads(kv_flat[:, H:], B, S)

    attn = _attention(q, k, v, pe, bm=128)
    attn = (attn.reshape(B, K, M, D).transpose(0, 2, 1, 3).reshape(T, H))

    h1 = _wo_ln(attn, h.reshape(T, H), _bf(Wo),
                ln1_g.reshape(1, H), ln1_b.reshape(1, H), min(1024, T))

    idx, prb = _router(h1, Wg, bg.reshape(1, E), min(1024, T))
    T2 = 2 * T
    ids_flat = idx.reshape(T2, 1)

    blk = min(1024, T2)
    rank, cnt = _rank(ids_flat, E, blk)
    off_full = jnp.concatenate(
        [jnp.zeros(1, jnp.int32), jnp.cumsum(cnt.reshape(E)).astype(jnp.int32)])
    pos = _pos(ids_flat, rank, off_full[:E].reshape(1, E), E, blk)
    pos_flat = pos.reshape(T2)
    pos0, pos1 = pos_flat[0::2], pos_flat[1::2]

    xg = _sc_scatter(h1, pos0, pos1)

    bm_g = min(512, T2)
    meta, ni = _gmm_meta(cnt, T2, bm_g, E)
    yg = _gmm(meta, ni, xg, _bf(W1), b1.reshape(E, 1, FF), _bf(W2),
              b2.reshape(E, 1, H), bm_g, min(768, FF))

    g0, g1 = _sc_gather(yg, pos0, pos1)

    h2 = _comb_ln(h1, g0, g1, prb[:, 0:1], prb[:, 1:2],
                  lnm_g.reshape(1, H), lnm_b.reshape(1, H),
                  ln2_g.reshape(1, H), ln2_b.reshape(1, H), min(1024, T))
    return h2.reshape(B, M, H)
